# baseline (speedup 1.0000x reference)
import functools

import jax
import jax.numpy as jnp
from jax import lax
from jax.experimental import pallas as pl
from jax.experimental.pallas import tpu as pltpu
from jax.experimental.pallas import tpu_sc as plsc

B, T, C = 32, 1024, 768
CHUNK = 32
NCH = T // CHUNK
NBUF = 4
NW = 32


def _loc_ed_body(img_hbm, idx_hbm, out_hbm, idx_v, bufs, gsems, ssems):
    cid = lax.axis_index("c")
    sid = lax.axis_index("s")
    b = sid * 2 + cid

    pltpu.sync_copy(idx_hbm, idx_v)

    gath = [None] * NBUF
    scat = [None] * NCH
    for j in range(NBUF):
        gath[j] = pltpu.async_copy(
            img_hbm.at[b].at[pl.ds(j * CHUNK, CHUNK)], bufs[j], gsems[j])
    for j in range(NCH):
        k = j % NBUF
        gath[k].wait()
        scat[j] = pltpu.async_copy(bufs[k], out_hbm.at[b].at[idx_v.at[j]],
                                   ssems[k])
        jw = j - (NBUF - 1)
        if jw >= 0 and jw + NBUF < NCH:
            scat[jw].wait()
            kw = jw % NBUF
            gath[kw] = pltpu.async_copy(
                img_hbm.at[b].at[pl.ds((jw + NBUF) * CHUNK, CHUNK)],
                bufs[kw], gsems[kw])
    for j in range(max(0, NCH - NBUF), NCH):
        scat[j].wait()


@functools.partial(
    pl.kernel,
    out_type=jax.ShapeDtypeStruct((B, T, C), jnp.float32),
    mesh=plsc.VectorSubcoreMesh(core_axis_name="c", subcore_axis_name="s"),
    scratch_types=[
        pltpu.VMEM((NCH, CHUNK), jnp.int32),
        [pltpu.VMEM((CHUNK, C), jnp.float32) for _ in range(NBUF)],
        [pltpu.SemaphoreType.DMA for _ in range(NBUF)],
        [pltpu.SemaphoreType.DMA for _ in range(NBUF)],
    ],
)
def _loc_ed_sc(img_hbm, idx_hbm, out_hbm, idx_v, bufs, gsems, ssems):
    _loc_ed_body(img_hbm, idx_hbm, out_hbm, idx_v, bufs, gsems, ssems)


def kernel(img, index_flat_inv):
    idx32 = index_flat_inv.astype(jnp.int32).reshape(NCH, CHUNK)
    return _loc_ed_sc(img, idx32)
